# baseline (speedup 1.0000x reference)
import functools

import jax
import jax.numpy as jnp
from jax.experimental import pallas as pl
from jax.experimental.pallas import tpu as pltpu

_B, _S, _D, _H, _E, _K = 2, 2048, 1024, 4096, 8, 2
_N = _B * _S

_TT = 2048
_HT = 512
_RT = 1024


def _router_body(x_ref, gnw_ref, noise_ref, idx_ref, gates_ref,
                 xb_ref, acc_ref):
    c = pl.program_id(0)
    xb = x_ref[...]
    gn = jnp.dot(xb, gnw_ref[...], preferred_element_type=jnp.float32)
    g = gn[:, :_E]
    sp = jax.nn.softplus(gn[:, _E:])
    part = (jnp.sum(g, axis=0, keepdims=True)
            + jnp.sum(sp, axis=0, keepdims=True) * noise_ref[...])
    xb_ref[...] = xb.astype(jnp.bfloat16)

    @pl.when(c == 0)
    def _():
        acc_ref[...] = part

    @pl.when(c != 0)
    def _():
        acc_ref[...] += part

    @pl.when(c == _N // _RT - 1)
    def _():
        ml = acc_ref[...] / _N
        iota = jax.lax.broadcasted_iota(jnp.int32, (1, _E), 1)
        v1 = jnp.max(ml)
        i1 = jnp.min(jnp.where(ml == v1, iota, _E))
        masked = jnp.where(iota == i1, -jnp.inf, ml)
        v2 = jnp.max(masked)
        i2 = jnp.min(jnp.where(masked == v2, iota, _E))
        e = jnp.exp(v2 - v1)
        idx_ref[0] = i1
        idx_ref[1] = i2
        gates_ref[0] = 1.0 / (1.0 + e)
        gates_ref[1] = e / (1.0 + e)


def _ffn_body(idx_sref, x_ref, w1a_ref, b1a_ref, w2a_ref, b2a_ref,
              w1b_ref, b1b_ref, w2b_ref, b2b_ref, gates_ref, o_ref):
    ht = pl.program_id(1)
    g0 = gates_ref[0]
    g1 = gates_ref[1]

    xb = x_ref[...]
    w1c = jnp.concatenate(
        [w1a_ref[0], w1b_ref[0]], axis=1).astype(jnp.bfloat16)
    b1c = jnp.concatenate([b1a_ref[0], b1b_ref[0]], axis=1)
    h = jnp.dot(xb, w1c, preferred_element_type=jnp.float32) + b1c
    h = jnp.maximum(h, 0.0).astype(jnp.bfloat16)
    w2c = jnp.concatenate(
        [w2a_ref[0] * g0, w2b_ref[0] * g1], axis=0).astype(jnp.bfloat16)
    acc = jnp.dot(h, w2c, preferred_element_type=jnp.float32)

    @pl.when(ht == 0)
    def _():
        o_ref[...] = acc + (g0 * b2a_ref[0] + g1 * b2b_ref[0])

    @pl.when(ht != 0)
    def _():
        o_ref[...] += acc


@jax.jit
def kernel(x, gate_w, noise_w, in_w, in_b, out_w, out_b, noise):
    x2 = x.reshape(_N, _D)

    idx, gates, xb = pl.pallas_call(
        _router_body,
        grid=(_N // _RT,),
        in_specs=[
            pl.BlockSpec((_RT, _D), lambda i: (i, 0)),
            pl.BlockSpec((_D, 2 * _E), lambda i: (0, 0)),
            pl.BlockSpec((1, _E), lambda i: (0, 0)),
        ],
        out_specs=[
            pl.BlockSpec(memory_space=pltpu.SMEM),
            pl.BlockSpec(memory_space=pltpu.SMEM),
            pl.BlockSpec((_RT, _D), lambda i: (i, 0)),
        ],
        out_shape=[
            jax.ShapeDtypeStruct((_K,), jnp.int32),
            jax.ShapeDtypeStruct((_K,), jnp.float32),
            jax.ShapeDtypeStruct((_N, _D), jnp.bfloat16),
        ],
        scratch_shapes=[pltpu.VMEM((1, _E), jnp.float32)],
        compiler_params=pltpu.CompilerParams(
            dimension_semantics=("arbitrary",),
        ),
    )(x2, jnp.concatenate([gate_w, noise_w], axis=1), noise.reshape(1, _E))

    in_b3 = in_b.reshape(_E, 1, _H)
    out_b3 = out_b.reshape(_E, 1, _D)

    grid = (_N // _TT, _H // _HT)
    out = pl.pallas_call(
        _ffn_body,
        grid_spec=pltpu.PrefetchScalarGridSpec(
            num_scalar_prefetch=1,
            grid=grid,
            in_specs=[
                pl.BlockSpec((_TT, _D), lambda t, h, idx: (t, 0)),
                pl.BlockSpec((1, _D, _HT), lambda t, h, idx: (idx[0], 0, h)),
                pl.BlockSpec((1, 1, _HT), lambda t, h, idx: (idx[0], 0, h)),
                pl.BlockSpec((1, _HT, _D), lambda t, h, idx: (idx[0], h, 0)),
                pl.BlockSpec((1, 1, _D), lambda t, h, idx: (idx[0], 0, 0)),
                pl.BlockSpec((1, _D, _HT), lambda t, h, idx: (idx[1], 0, h)),
                pl.BlockSpec((1, 1, _HT), lambda t, h, idx: (idx[1], 0, h)),
                pl.BlockSpec((1, _HT, _D), lambda t, h, idx: (idx[1], h, 0)),
                pl.BlockSpec((1, 1, _D), lambda t, h, idx: (idx[1], 0, 0)),
                pl.BlockSpec(memory_space=pltpu.SMEM),
            ],
            out_specs=pl.BlockSpec((_TT, _D), lambda t, h, idx: (t, 0)),
        ),
        out_shape=jax.ShapeDtypeStruct((_N, _D), jnp.float32),
        compiler_params=pltpu.CompilerParams(
            dimension_semantics=("arbitrary", "arbitrary"),
        ),
    )(idx, xb, in_w, in_b3, out_w, out_b3, in_w, in_b3, out_w, out_b3,
      gates)

    return out.reshape(_B, _S, _D)


# speedup vs baseline: 1.0424x; 1.0424x over previous
import functools

import jax
import jax.numpy as jnp
from jax.experimental import pallas as pl
from jax.experimental.pallas import tpu as pltpu

_B, _S, _D, _H, _E, _K = 2, 2048, 1024, 4096, 8, 2
_N = _B * _S

_TT = 1024
_HT = 1024
_RT = 1024


def _router_body(x_ref, gnw_ref, noise_ref, idx_ref, gates_ref,
                 xb_ref, acc_ref):
    c = pl.program_id(0)
    xb = x_ref[...]
    gn = jnp.dot(xb, gnw_ref[...], preferred_element_type=jnp.float32)
    g = gn[:, :_E]
    sp = jax.nn.softplus(gn[:, _E:])
    part = (jnp.sum(g, axis=0, keepdims=True)
            + jnp.sum(sp, axis=0, keepdims=True) * noise_ref[...])
    xb_ref[...] = xb.astype(jnp.bfloat16)

    @pl.when(c == 0)
    def _():
        acc_ref[...] = part

    @pl.when(c != 0)
    def _():
        acc_ref[...] += part

    @pl.when(c == _N // _RT - 1)
    def _():
        ml = acc_ref[...] / _N
        iota = jax.lax.broadcasted_iota(jnp.int32, (1, _E), 1)
        v1 = jnp.max(ml)
        i1 = jnp.min(jnp.where(ml == v1, iota, _E))
        masked = jnp.where(iota == i1, -jnp.inf, ml)
        v2 = jnp.max(masked)
        i2 = jnp.min(jnp.where(masked == v2, iota, _E))
        e = jnp.exp(v2 - v1)
        idx_ref[0] = i1
        idx_ref[1] = i2
        gates_ref[0] = 1.0 / (1.0 + e)
        gates_ref[1] = e / (1.0 + e)


def _ffn_body(idx_sref, x_ref, w1a_ref, b1a_ref, w2a_ref, b2a_ref,
              w1b_ref, b1b_ref, w2b_ref, b2b_ref, gates_ref, o_ref):
    ht = pl.program_id(1)
    g0 = gates_ref[0]
    g1 = gates_ref[1]

    xb = x_ref[...]
    w1c = jnp.concatenate(
        [w1a_ref[0], w1b_ref[0]], axis=1).astype(jnp.bfloat16)
    b1c = jnp.concatenate([b1a_ref[0], b1b_ref[0]], axis=1)
    h = jnp.dot(xb, w1c, preferred_element_type=jnp.float32) + b1c
    h = jnp.maximum(h, 0.0).astype(jnp.bfloat16)
    w2c = jnp.concatenate(
        [w2a_ref[0] * g0, w2b_ref[0] * g1], axis=0).astype(jnp.bfloat16)
    acc = jnp.dot(h, w2c, preferred_element_type=jnp.float32)

    @pl.when(ht == 0)
    def _():
        o_ref[...] = acc + (g0 * b2a_ref[0] + g1 * b2b_ref[0])

    @pl.when(ht != 0)
    def _():
        o_ref[...] += acc


@jax.jit
def kernel(x, gate_w, noise_w, in_w, in_b, out_w, out_b, noise):
    x2 = x.reshape(_N, _D)

    idx, gates, xb = pl.pallas_call(
        _router_body,
        grid=(_N // _RT,),
        in_specs=[
            pl.BlockSpec((_RT, _D), lambda i: (i, 0)),
            pl.BlockSpec((_D, 2 * _E), lambda i: (0, 0)),
            pl.BlockSpec((1, _E), lambda i: (0, 0)),
        ],
        out_specs=[
            pl.BlockSpec(memory_space=pltpu.SMEM),
            pl.BlockSpec(memory_space=pltpu.SMEM),
            pl.BlockSpec((_RT, _D), lambda i: (i, 0)),
        ],
        out_shape=[
            jax.ShapeDtypeStruct((_K,), jnp.int32),
            jax.ShapeDtypeStruct((_K,), jnp.float32),
            jax.ShapeDtypeStruct((_N, _D), jnp.bfloat16),
        ],
        scratch_shapes=[pltpu.VMEM((1, _E), jnp.float32)],
        compiler_params=pltpu.CompilerParams(
            dimension_semantics=("arbitrary",),
        ),
    )(x2, jnp.concatenate([gate_w, noise_w], axis=1), noise.reshape(1, _E))

    in_b3 = in_b.reshape(_E, 1, _H)
    out_b3 = out_b.reshape(_E, 1, _D)

    grid = (_N // _TT, _H // _HT)
    out = pl.pallas_call(
        _ffn_body,
        grid_spec=pltpu.PrefetchScalarGridSpec(
            num_scalar_prefetch=1,
            grid=grid,
            in_specs=[
                pl.BlockSpec((_TT, _D), lambda t, h, idx: (t, 0)),
                pl.BlockSpec((1, _D, _HT), lambda t, h, idx: (idx[0], 0, h)),
                pl.BlockSpec((1, 1, _HT), lambda t, h, idx: (idx[0], 0, h)),
                pl.BlockSpec((1, _HT, _D), lambda t, h, idx: (idx[0], h, 0)),
                pl.BlockSpec((1, 1, _D), lambda t, h, idx: (idx[0], 0, 0)),
                pl.BlockSpec((1, _D, _HT), lambda t, h, idx: (idx[1], 0, h)),
                pl.BlockSpec((1, 1, _HT), lambda t, h, idx: (idx[1], 0, h)),
                pl.BlockSpec((1, _HT, _D), lambda t, h, idx: (idx[1], h, 0)),
                pl.BlockSpec((1, 1, _D), lambda t, h, idx: (idx[1], 0, 0)),
                pl.BlockSpec(memory_space=pltpu.SMEM),
            ],
            out_specs=pl.BlockSpec((_TT, _D), lambda t, h, idx: (t, 0)),
        ),
        out_shape=jax.ShapeDtypeStruct((_N, _D), jnp.float32),
        compiler_params=pltpu.CompilerParams(
            dimension_semantics=("arbitrary", "arbitrary"),
        ),
    )(idx, xb, in_w, in_b3, out_w, out_b3, in_w, in_b3, out_w, out_b3,
      gates)

    return out.reshape(_B, _S, _D)
